# Initial kernel scaffold; baseline (speedup 1.0000x reference)
import jax
import jax.numpy as jnp
from jax.experimental import pallas as pl


def kernel(node_features, node_trans, edge_features, edge_index, k, ln_gamma, ln_beta, W_nl, b_nl, W_nr, b_nr, W_bg, b_bg, W_db, b_db, W_tb, W_q, b_q, W_kv, b_kv, W_gate, b_gate, W_out, b_out):
    raise NotImplementedError("write your pallas kernel here")


# speedup vs baseline: 67.2090x; 67.2090x over previous
import functools

import jax
import jax.numpy as jnp
import numpy as np
from jax import lax
from jax.experimental import pallas as pl
from jax.experimental.pallas import tpu as pltpu
from jax.experimental.pallas import tpu_sc as plsc

N_NODES = 10000
DEG = 16
E_EDGES = N_NODES * DEG
C_Z = 128
H = 4
D_H = C_Z // H
NUM_RBF = 64
TBL_W = 128

G = 8
BLK_E = G * DEG
N_BLOCKS = E_EDGES // BLK_E

SC_NC = 2
SC_NS = 16
SC_W = SC_NC * SC_NS
SC_CHUNK = 128
SC_CPW = 40
E_PAD = SC_W * SC_CPW * SC_CHUNK


def _table_body(nt_ref, nf_ref, w_ref, b_ref, out_ref):
    nf = nf_ref[...]
    nlr = jnp.dot(nf, w_ref[...], preferred_element_type=jnp.float32) + b_ref[...]
    nt = nt_ref[...]
    z5 = jnp.zeros((nt.shape[0], 5), jnp.float32)
    zpad = jnp.zeros((nt.shape[0], TBL_W - 24), jnp.float32)
    out_ref[...] = jnp.concatenate([nt, z5, nlr, zpad], axis=1)


def _build_table(node_trans, node_features, W_lr, b_lr):
    blk = 2000
    grid = (N_NODES // blk,)
    return pl.pallas_call(
        _table_body,
        grid=grid,
        in_specs=[
            pl.BlockSpec((blk, 3), lambda i: (i, 0)),
            pl.BlockSpec((blk, C_Z), lambda i: (i, 0)),
            pl.BlockSpec((C_Z, 16), lambda i: (0, 0)),
            pl.BlockSpec((1, 16), lambda i: (0, 0)),
        ],
        out_specs=pl.BlockSpec((blk, TBL_W), lambda i: (i, 0)),
        out_shape=jax.ShapeDtypeStruct((N_NODES, TBL_W), jnp.float32),
    )(node_trans, node_features, W_lr, b_lr)


def _sc_gather_body(table_hbm, idx_hbm, out_hbm, idx_v, rows_v, sem):
    wid = lax.axis_index("s") * SC_NC + lax.axis_index("c")

    def chunk(c, carry):
        base = wid * (SC_CPW * SC_CHUNK) + c * SC_CHUNK
        pltpu.sync_copy(idx_hbm.at[pl.ds(base, SC_CHUNK)], idx_v)
        pltpu.async_copy(table_hbm.at[idx_v], rows_v, sem).wait()
        pltpu.sync_copy(rows_v, out_hbm.at[pl.ds(base, SC_CHUNK)])
        return carry

    lax.fori_loop(0, SC_CPW, chunk, 0)


def _sc_gather(table, idx_pad):
    f = functools.partial(
        pl.kernel,
        out_type=jax.ShapeDtypeStruct((E_PAD, TBL_W), jnp.float32),
        mesh=plsc.VectorSubcoreMesh(
            core_axis_name="c", subcore_axis_name="s", num_cores=SC_NC),
        scratch_types=[
            pltpu.VMEM((SC_CHUNK,), jnp.int32),
            pltpu.VMEM((SC_CHUNK, TBL_W), jnp.float32),
            pltpu.SemaphoreType.DMA,
        ],
    )(_sc_gather_body)
    return f(table, idx_pad)


def _main_body(gat_ref, ef_ref, lng_ref, lnb_ref, wq_ref, bq_ref, wkv_ref,
               bkv_ref, wg_ref, bg_ref, wbg_ref, bbg_ref, wdb_ref, bdb_ref,
               wtb_ref, wo_ref, bo_ref, out_ref):
    f32 = jnp.float32
    ef0 = ef_ref[...]
    mu_ = jnp.mean(ef0, axis=-1, keepdims=True)
    xc = ef0 - mu_
    var = jnp.mean(xc * xc, axis=-1, keepdims=True)
    ef = xc / jnp.sqrt(var + 1e-5) * lng_ref[...] + lnb_ref[...]

    q = jnp.dot(ef, wq_ref[...], preferred_element_type=f32) + bq_ref[...]
    kv = jnp.dot(ef, wkv_ref[...], preferred_element_type=f32) + bkv_ref[...]
    kk = kv[:, :C_Z]
    v = kv[:, C_Z:]
    gate = jax.nn.sigmoid(
        jnp.dot(ef, wg_ref[...], preferred_element_type=f32) + bg_ref[...])

    gat = gat_ref[...]
    t3 = gat[:, 0:3].reshape(G, DEG, 3)
    nl = gat[:, 8:16]
    nr = gat[:, 16:24]

    diff = t3[:, :, None, :] - t3[:, None, :, :]
    d2 = jnp.sum(diff * diff, axis=-1)
    r16 = lax.broadcasted_iota(jnp.int32, (DEG, DEG), 0)
    c16 = lax.broadcasted_iota(jnp.int32, (DEG, DEG), 1)
    d2 = jnp.where((r16 == c16)[None], jnp.inf, d2)

    d2i = d2[:, :, :, None]
    d2l = d2[:, :, None, :]
    ii = lax.broadcasted_iota(jnp.int32, (1, 1, DEG, DEG), 2)
    li = lax.broadcasted_iota(jnp.int32, (1, 1, DEG, DEG), 3)
    beats = (d2l < d2i) | ((d2l == d2i) & (li < ii))
    cnt = jnp.sum(beats.astype(jnp.int32), axis=-1)
    sel = cnt < 8

    a64 = jnp.repeat(nl, 8, axis=1)
    b64 = jnp.tile(nr, (1, 8))
    pair = (a64.reshape(G, DEG, 1, 64) * b64.reshape(G, 1, DEG, 64))
    pair = pair.reshape(G * DEG * DEG, 64)
    e3g = jnp.dot(pair, wbg_ref[...], preferred_element_type=f32) + bbg_ref[...]

    dist = jnp.sqrt(jnp.sum((diff + 1e-8) ** 2, axis=-1))
    d1 = dist.reshape(G * DEG * DEG, 1)
    mu_rbf = lax.broadcasted_iota(jnp.int32, (1, NUM_RBF), 1).astype(f32) * (
        20.0 / (NUM_RBF - 1))
    sigma = 20.0 / NUM_RBF
    rbf = jnp.exp(-(((d1 - mu_rbf) / sigma) ** 2))
    db = jnp.dot(rbf, wdb_ref[...], preferred_element_type=f32) + bdb_ref[...]

    zb = jax.nn.sigmoid(e3g) * db
    bias = jnp.dot(zb, wtb_ref[...], preferred_element_type=f32)
    bias_g = bias.reshape(G, DEG, DEG, H)

    scale = 1.0 / np.sqrt(C_Z)
    outs = []
    for h in range(H):
        qh = q[:, h * D_H:(h + 1) * D_H].reshape(G, DEG, D_H)
        kh = kk[:, h * D_H:(h + 1) * D_H].reshape(G, DEG, D_H)
        vh = v[:, h * D_H:(h + 1) * D_H].reshape(G, DEG, D_H)
        lg = lax.dot_general(qh, kh, (((2,), (2,)), ((0,), (0,))),
                             preferred_element_type=f32) * scale
        lg = lg + bias_g[:, :, :, h]
        lg = jnp.where(sel, lg, -jnp.inf)
        mx = jnp.max(lg, axis=-1, keepdims=True)
        ex = jnp.exp(lg - mx)
        sm = jnp.sum(ex, axis=-1, keepdims=True)
        p = ex / (sm + 1e-16)
        uh = lax.dot_general(p, vh, (((2,), (1,)), ((0,), (0,))),
                             preferred_element_type=f32)
        outs.append(uh.reshape(BLK_E, D_H))
    upd = jnp.concatenate(outs, axis=1)
    out = jnp.dot(upd * gate, wo_ref[...], preferred_element_type=f32)
    out_ref[...] = out + bo_ref[...]


def _main(gathered, edge_features, ln_gamma, ln_beta, W_q, b_q, W_kv, b_kv,
          W_gate, b_gate, W_bg, b_bg, W_db, b_db, W_tb, W_out, b_out):
    full = lambda shape: pl.BlockSpec(shape, lambda i: tuple(0 for _ in shape))
    return pl.pallas_call(
        _main_body,
        grid=(N_BLOCKS,),
        in_specs=[
            pl.BlockSpec((BLK_E, TBL_W), lambda i: (i, 0)),
            pl.BlockSpec((BLK_E, C_Z), lambda i: (i, 0)),
            full((1, C_Z)), full((1, C_Z)),
            full((C_Z, C_Z)), full((1, C_Z)),
            full((C_Z, 2 * C_Z)), full((1, 2 * C_Z)),
            full((C_Z, C_Z)), full((1, C_Z)),
            full((64, C_Z)), full((1, C_Z)),
            full((NUM_RBF, C_Z)), full((1, C_Z)),
            full((C_Z, H)),
            full((C_Z, C_Z)), full((1, C_Z)),
        ],
        out_specs=pl.BlockSpec((BLK_E, C_Z), lambda i: (i, 0)),
        out_shape=jax.ShapeDtypeStruct((E_EDGES, C_Z), jnp.float32),
    )(gathered, edge_features, ln_gamma, ln_beta, W_q, b_q, W_kv, b_kv,
      W_gate, b_gate, W_bg, b_bg, W_db, b_db, W_tb, W_out, b_out)


def kernel(node_features, node_trans, edge_features, edge_index, k, ln_gamma,
           ln_beta, W_nl, b_nl, W_nr, b_nr, W_bg, b_bg, W_db, b_db, W_tb,
           W_q, b_q, W_kv, b_kv, W_gate, b_gate, W_out, b_out):
    W_lr = jnp.concatenate([W_nl, W_nr], axis=1)
    b_lr = jnp.concatenate([b_nl, b_nr])[None, :]
    table = _build_table(node_trans, node_features, W_lr, b_lr)
    src = jnp.pad(edge_index[0].astype(jnp.int32), (0, E_PAD - E_EDGES))
    gathered = _sc_gather(table, src)
    row = lambda x: x[None, :]
    return _main(gathered, edge_features, row(ln_gamma), row(ln_beta),
                 W_q, row(b_q), W_kv, row(b_kv), W_gate, row(b_gate),
                 W_bg, row(b_bg), W_db, row(b_db), W_tb, W_out, row(b_out))


# speedup vs baseline: 132.4452x; 1.9706x over previous
import functools

import jax
import jax.numpy as jnp
import numpy as np
from jax import lax
from jax.experimental import pallas as pl
from jax.experimental.pallas import tpu as pltpu
from jax.experimental.pallas import tpu_sc as plsc

N_NODES = 10000
DEG = 16
E_EDGES = N_NODES * DEG
C_Z = 128
H = 4
D_H = C_Z // H
NUM_RBF = 64
TBL_W = 128

G = 8
BLK_E = G * DEG
N_BLOCKS = E_EDGES // BLK_E

SC_NC = 2
SC_NS = 16
SC_W = SC_NC * SC_NS
SC_CHUNK = 128
SC_CPW = 40
E_PAD = SC_W * SC_CPW * SC_CHUNK


def _table_body(nt_ref, nf_ref, w_ref, b_ref, out_ref):
    nf = nf_ref[...]
    nlr = jnp.dot(nf, w_ref[...], preferred_element_type=jnp.float32) + b_ref[...]
    nt = nt_ref[...]
    z5 = jnp.zeros((nt.shape[0], 5), jnp.float32)
    zpad = jnp.zeros((nt.shape[0], TBL_W - 24), jnp.float32)
    out_ref[...] = jnp.concatenate([nt, z5, nlr, zpad], axis=1)


def _build_table(node_trans, node_features, W_lr, b_lr):
    blk = 2000
    grid = (N_NODES // blk,)
    return pl.pallas_call(
        _table_body,
        grid=grid,
        in_specs=[
            pl.BlockSpec((blk, 3), lambda i: (i, 0)),
            pl.BlockSpec((blk, C_Z), lambda i: (i, 0)),
            pl.BlockSpec((C_Z, 16), lambda i: (0, 0)),
            pl.BlockSpec((1, 16), lambda i: (0, 0)),
        ],
        out_specs=pl.BlockSpec((blk, TBL_W), lambda i: (i, 0)),
        out_shape=jax.ShapeDtypeStruct((N_NODES, TBL_W), jnp.float32),
    )(node_trans, node_features, W_lr, b_lr)


def _sc_gather_body(table_hbm, idx_hbm, out_hbm, idx_v, rows_v, sem):
    wid = lax.axis_index("s") * SC_NC + lax.axis_index("c")

    def chunk(c, carry):
        base = wid * (SC_CPW * SC_CHUNK) + c * SC_CHUNK
        pltpu.sync_copy(idx_hbm.at[pl.ds(base, SC_CHUNK)], idx_v)
        pltpu.async_copy(table_hbm.at[idx_v], rows_v, sem).wait()
        pltpu.sync_copy(rows_v, out_hbm.at[pl.ds(base, SC_CHUNK)])
        return carry

    lax.fori_loop(0, SC_CPW, chunk, 0)


def _sc_gather(table, idx_pad):
    f = functools.partial(
        pl.kernel,
        out_type=jax.ShapeDtypeStruct((E_PAD, TBL_W), jnp.float32),
        mesh=plsc.VectorSubcoreMesh(
            core_axis_name="c", subcore_axis_name="s", num_cores=SC_NC),
        scratch_types=[
            pltpu.VMEM((SC_CHUNK,), jnp.int32),
            pltpu.VMEM((SC_CHUNK, TBL_W), jnp.float32),
            pltpu.SemaphoreType.DMA,
        ],
    )(_sc_gather_body)
    return f(table, idx_pad)


def _main_body(gat_ref, ef_ref, lng_ref, lnb_ref, wq_ref, bq_ref, wkv_ref,
               bkv_ref, wg_ref, bg_ref, wbg_ref, bbg_ref, wdb_ref, bdb_ref,
               wtb_ref, wo_ref, bo_ref, out_ref):
    f32 = jnp.float32
    ef0 = ef_ref[...]
    mu_ = jnp.mean(ef0, axis=-1, keepdims=True)
    xc = ef0 - mu_
    var = jnp.mean(xc * xc, axis=-1, keepdims=True)
    ef = xc / jnp.sqrt(var + 1e-5) * lng_ref[...] + lnb_ref[...]

    q = jnp.dot(ef, wq_ref[...], preferred_element_type=f32) + bq_ref[...]
    kv = jnp.dot(ef, wkv_ref[...], preferred_element_type=f32) + bkv_ref[...]
    kk = kv[:, :C_Z]
    v = kv[:, C_Z:]
    gate = jax.nn.sigmoid(
        jnp.dot(ef, wg_ref[...], preferred_element_type=f32) + bg_ref[...])

    gat = gat_ref[...]

    def nbr16(col):
        cg = col.reshape(G, DEG)
        return jnp.broadcast_to(cg[:, None, :], (G, DEG, DEG)).reshape(
            BLK_E, DEG)

    tx, ty, tz = gat[:, 0:1], gat[:, 1:2], gat[:, 2:3]
    dx = tx - nbr16(tx)
    dy = ty - nbr16(ty)
    dz = tz - nbr16(tz)
    d2 = dx * dx + dy * dy + dz * dz
    rowm = lax.broadcasted_iota(jnp.int32, (BLK_E, DEG), 0) % DEG
    lane = lax.broadcasted_iota(jnp.int32, (BLK_E, DEG), 1)
    d2 = jnp.where(rowm == lane, jnp.inf, d2)

    d2_i = d2[:, None, :]
    d2_l = d2[:, :, None]
    li = lax.broadcasted_iota(jnp.int32, (1, DEG, DEG), 1)
    ii = lax.broadcasted_iota(jnp.int32, (1, DEG, DEG), 2)
    beats = (d2_l < d2_i) | ((d2_l == d2_i) & (li < ii))
    cnt = jnp.sum(beats.astype(jnp.int32), axis=1)
    rg = lax.broadcasted_iota(jnp.int32, (BLK_E, BLK_E), 0) // DEG
    cg_ = lax.broadcasted_iota(jnp.int32, (BLK_E, BLK_E), 1) // DEG
    sel128 = (jnp.tile(cnt, (1, G)) < 8) & (rg == cg_)

    nl = gat[:, 8:16]
    nr = gat[:, 16:24]
    a64 = jnp.repeat(nl, 8, axis=1)
    b64 = jnp.tile(nr, (1, 8))
    a2048 = jnp.broadcast_to(a64[:, None, :], (BLK_E, DEG, 64)).reshape(
        BLK_E * DEG, 64)
    b2048 = jnp.broadcast_to(
        b64.reshape(G, 1, DEG, 64), (G, DEG, DEG, 64)).reshape(BLK_E * DEG, 64)
    pair = a2048 * b2048
    e3g = jnp.dot(pair, wbg_ref[...], preferred_element_type=f32) + bbg_ref[...]

    def self2048(col):
        return jnp.broadcast_to(col[:, None, :], (BLK_E, DEG, 1)).reshape(
            BLK_E * DEG, 1)

    def nbr2048(col):
        return jnp.broadcast_to(
            col.reshape(G, 1, DEG, 1), (G, DEG, DEG, 1)).reshape(
                BLK_E * DEG, 1)

    px = self2048(tx) - nbr2048(tx)
    py = self2048(ty) - nbr2048(ty)
    pz = self2048(tz) - nbr2048(tz)
    d1 = jnp.sqrt((px + 1e-8) ** 2 + (py + 1e-8) ** 2 + (pz + 1e-8) ** 2)
    mu_rbf = lax.broadcasted_iota(jnp.int32, (1, NUM_RBF), 1).astype(f32) * (
        20.0 / (NUM_RBF - 1))
    sigma = 20.0 / NUM_RBF
    rbf = jnp.exp(-(((d1 - mu_rbf) / sigma) ** 2))
    db = jnp.dot(rbf, wdb_ref[...], preferred_element_type=f32) + bdb_ref[...]

    zb = jax.nn.sigmoid(e3g) * db
    bias4 = jnp.dot(zb, wtb_ref[...], preferred_element_type=f32)
    b64h = jnp.swapaxes(bias4.reshape(BLK_E, DEG, H), 1, 2).reshape(
        BLK_E, DEG * H)

    scale = 1.0 / np.sqrt(C_Z)
    outs = []
    for h in range(H):
        qh = q[:, h * D_H:(h + 1) * D_H]
        kh = kk[:, h * D_H:(h + 1) * D_H]
        vh = v[:, h * D_H:(h + 1) * D_H]
        lg = lax.dot_general(qh, kh, (((1,), (1,)), ((), ())),
                             preferred_element_type=f32) * scale
        lg = lg + jnp.tile(b64h[:, h * DEG:(h + 1) * DEG], (1, G))
        lg = jnp.where(sel128, lg, -jnp.inf)
        mx = jnp.max(lg, axis=-1, keepdims=True)
        ex = jnp.exp(lg - mx)
        sm = jnp.sum(ex, axis=-1, keepdims=True)
        p = ex / (sm + 1e-16)
        outs.append(jnp.dot(p, vh, preferred_element_type=f32))
    upd = jnp.concatenate(outs, axis=1)
    out = jnp.dot(upd * gate, wo_ref[...], preferred_element_type=f32)
    out_ref[...] = out + bo_ref[...]


def _main(gathered, edge_features, ln_gamma, ln_beta, W_q, b_q, W_kv, b_kv,
          W_gate, b_gate, W_bg, b_bg, W_db, b_db, W_tb, W_out, b_out):
    full = lambda shape: pl.BlockSpec(shape, lambda i: tuple(0 for _ in shape))
    return pl.pallas_call(
        _main_body,
        grid=(N_BLOCKS,),
        in_specs=[
            pl.BlockSpec((BLK_E, TBL_W), lambda i: (i, 0)),
            pl.BlockSpec((BLK_E, C_Z), lambda i: (i, 0)),
            full((1, C_Z)), full((1, C_Z)),
            full((C_Z, C_Z)), full((1, C_Z)),
            full((C_Z, 2 * C_Z)), full((1, 2 * C_Z)),
            full((C_Z, C_Z)), full((1, C_Z)),
            full((64, C_Z)), full((1, C_Z)),
            full((NUM_RBF, C_Z)), full((1, C_Z)),
            full((C_Z, H)),
            full((C_Z, C_Z)), full((1, C_Z)),
        ],
        out_specs=pl.BlockSpec((BLK_E, C_Z), lambda i: (i, 0)),
        out_shape=jax.ShapeDtypeStruct((E_EDGES, C_Z), jnp.float32),
    )(gathered, edge_features, ln_gamma, ln_beta, W_q, b_q, W_kv, b_kv,
      W_gate, b_gate, W_bg, b_bg, W_db, b_db, W_tb, W_out, b_out)


def kernel(node_features, node_trans, edge_features, edge_index, k, ln_gamma,
           ln_beta, W_nl, b_nl, W_nr, b_nr, W_bg, b_bg, W_db, b_db, W_tb,
           W_q, b_q, W_kv, b_kv, W_gate, b_gate, W_out, b_out):
    W_lr = jnp.concatenate([W_nl, W_nr], axis=1)
    b_lr = jnp.concatenate([b_nl, b_nr])[None, :]
    table = _build_table(node_trans, node_features, W_lr, b_lr)
    src = jnp.pad(edge_index[0].astype(jnp.int32), (0, E_PAD - E_EDGES))
    gathered = _sc_gather(table, src)
    row = lambda x: x[None, :]
    return _main(gathered, edge_features, row(ln_gamma), row(ln_beta),
                 W_q, row(b_q), W_kv, row(b_kv), W_gate, row(b_gate),
                 W_bg, row(b_bg), W_db, row(b_db), W_tb, W_out, row(b_out))


# speedup vs baseline: 170.7721x; 1.2894x over previous
import functools

import jax
import jax.numpy as jnp
import numpy as np
from jax import lax
from jax.experimental import pallas as pl
from jax.experimental.pallas import tpu as pltpu
from jax.experimental.pallas import tpu_sc as plsc

N_NODES = 10000
DEG = 16
E_EDGES = N_NODES * DEG
C_Z = 128
H = 4
D_H = C_Z // H
NUM_RBF = 64
TBL_W = 128

G = 8
BLK_E = G * DEG
N_BLOCKS = E_EDGES // BLK_E

SC_NC = 2
SC_NS = 16
SC_W = SC_NC * SC_NS
SC_CHUNK = 128
SC_CPW = 40
E_PAD = SC_W * SC_CPW * SC_CHUNK


def _table_body(nt_ref, nf_ref, w_ref, b_ref, out_ref):
    nf = nf_ref[...]
    nlr = jnp.dot(nf, w_ref[...], preferred_element_type=jnp.float32) + b_ref[...]
    nt = nt_ref[...]
    z5 = jnp.zeros((nt.shape[0], 5), jnp.float32)
    zpad = jnp.zeros((nt.shape[0], TBL_W - 24), jnp.float32)
    out_ref[...] = jnp.concatenate([nt, z5, nlr, zpad], axis=1)


def _build_table(node_trans, node_features, W_lr, b_lr):
    blk = 2000
    grid = (N_NODES // blk,)
    return pl.pallas_call(
        _table_body,
        grid=grid,
        in_specs=[
            pl.BlockSpec((blk, 3), lambda i: (i, 0)),
            pl.BlockSpec((blk, C_Z), lambda i: (i, 0)),
            pl.BlockSpec((C_Z, 16), lambda i: (0, 0)),
            pl.BlockSpec((1, 16), lambda i: (0, 0)),
        ],
        out_specs=pl.BlockSpec((blk, TBL_W), lambda i: (i, 0)),
        out_shape=jax.ShapeDtypeStruct((N_NODES, TBL_W), jnp.float32),
    )(node_trans, node_features, W_lr, b_lr)


def _sc_gather_body(table_hbm, idx_hbm, out_hbm, idx_v, rows_v, sem):
    wid = lax.axis_index("s") * SC_NC + lax.axis_index("c")

    def chunk(c, carry):
        base = wid * (SC_CPW * SC_CHUNK) + c * SC_CHUNK
        pltpu.sync_copy(idx_hbm.at[pl.ds(base, SC_CHUNK)], idx_v)
        pltpu.async_copy(table_hbm.at[idx_v], rows_v, sem).wait()
        pltpu.sync_copy(rows_v, out_hbm.at[pl.ds(base, SC_CHUNK)])
        return carry

    lax.fori_loop(0, SC_CPW, chunk, 0)


def _sc_gather(table, idx_pad):
    f = functools.partial(
        pl.kernel,
        out_type=jax.ShapeDtypeStruct((E_PAD, TBL_W), jnp.float32),
        mesh=plsc.VectorSubcoreMesh(
            core_axis_name="c", subcore_axis_name="s", num_cores=SC_NC),
        scratch_types=[
            pltpu.VMEM((SC_CHUNK,), jnp.int32),
            pltpu.VMEM((SC_CHUNK, TBL_W), jnp.float32),
            pltpu.SemaphoreType.DMA,
        ],
    )(_sc_gather_body)
    return f(table, idx_pad)


def _main_body(gat_ref, ef_ref, rrep_ref, trep_ref, neg_ref, ea_ref, eb_ref,
               lng_ref,
               lnb_ref, wq_ref, bq_ref, wkv_ref, bkv_ref, wg_ref, bg_ref,
               wbg_ref, bbg_ref, wdb_ref, bdb_ref, wtb_ref, wo_ref, bo_ref,
               out_ref):
    f32 = jnp.float32
    ef0 = ef_ref[...]
    mu_ = jnp.mean(ef0, axis=-1, keepdims=True)
    xc = ef0 - mu_
    var = jnp.mean(xc * xc, axis=-1, keepdims=True)
    ef = xc / jnp.sqrt(var + 1e-5) * lng_ref[...] + lnb_ref[...]

    q = jnp.dot(ef, wq_ref[...], preferred_element_type=f32) + bq_ref[...]
    kv = jnp.dot(ef, wkv_ref[...], preferred_element_type=f32) + bkv_ref[...]
    kk = kv[:, :C_Z]
    v = kv[:, C_Z:]
    gate = jax.nn.sigmoid(
        jnp.dot(ef, wg_ref[...], preferred_element_type=f32) + bg_ref[...])

    gat = gat_ref[...]

    def nbr16(col):
        cg = col.reshape(G, DEG)
        return jnp.broadcast_to(cg[:, None, :], (G, DEG, DEG)).reshape(
            BLK_E, DEG)

    tx, ty, tz = gat[:, 0:1], gat[:, 1:2], gat[:, 2:3]
    dx = tx - nbr16(tx)
    dy = ty - nbr16(ty)
    dz = tz - nbr16(tz)
    d2 = dx * dx + dy * dy + dz * dz
    rowm = lax.broadcasted_iota(jnp.int32, (BLK_E, DEG), 0) % DEG
    lane = lax.broadcasted_iota(jnp.int32, (BLK_E, DEG), 1)
    d2 = jnp.where(rowm == lane, jnp.inf, d2)

    d2_i = d2[:, None, :]
    d2_l = d2[:, :, None]
    li = lax.broadcasted_iota(jnp.int32, (1, DEG, DEG), 1)
    ii = lax.broadcasted_iota(jnp.int32, (1, DEG, DEG), 2)
    beats = (d2_l < d2_i) | ((d2_l == d2_i) & (li < ii))
    cnt = jnp.sum(beats.astype(f32), axis=1)
    sel128 = jnp.tile(cnt, (1, G)) < 8

    gat24 = gat[:, 0:24]
    lhsA = jnp.dot(gat24, ea_ref[...], preferred_element_type=f32)
    lhsB = jnp.dot(gat24, eb_ref[...], preferred_element_type=f32)
    aside = jnp.dot(rrep_ref[...], lhsA, preferred_element_type=f32)
    bside = jnp.dot(trep_ref[...], lhsB, preferred_element_type=f32)
    pair = aside[:, :64] * bside[:, :64]
    e3g = jnp.dot(pair, wbg_ref[...], preferred_element_type=f32) + bbg_ref[...]

    px = (aside[:, 64:65] - bside[:, 64:65]) + 1e-8
    py = (aside[:, 65:66] - bside[:, 65:66]) + 1e-8
    pz = (aside[:, 66:67] - bside[:, 66:67]) + 1e-8
    d1 = jnp.sqrt(px * px + py * py + pz * pz)
    mu_rbf = lax.broadcasted_iota(jnp.int32, (1, NUM_RBF), 1).astype(f32) * (
        20.0 / (NUM_RBF - 1))
    sigma = 20.0 / NUM_RBF
    rbf = jnp.exp(-(((d1 - mu_rbf) / sigma) ** 2))
    db = jnp.dot(rbf, wdb_ref[...], preferred_element_type=f32) + bdb_ref[...]

    zb = jax.nn.sigmoid(e3g) * db
    bias4 = jnp.dot(zb, wtb_ref[...], preferred_element_type=f32)
    b64h = jnp.swapaxes(bias4.reshape(BLK_E, DEG, H), 1, 2).reshape(
        BLK_E, DEG * H)

    neg = neg_ref[...]
    outs = []
    for h in range(H):
        qh = q[:, h * D_H:(h + 1) * D_H]
        kh = kk[:, h * D_H:(h + 1) * D_H]
        vh = v[:, h * D_H:(h + 1) * D_H]
        lg = lax.dot_general(qh, kh, (((1,), (1,)), ((), ())),
                             preferred_element_type=f32)
        lg = lg + jnp.tile(b64h[:, h * DEG:(h + 1) * DEG], (1, G)) + neg
        lg = jnp.where(sel128, lg, -jnp.inf)
        mx = jnp.max(lg, axis=-1, keepdims=True)
        ex = jnp.exp(lg - mx)
        sm = jnp.sum(ex, axis=-1, keepdims=True)
        p = ex / (sm + 1e-16)
        outs.append(jnp.dot(p, vh, preferred_element_type=f32))
    upd = jnp.concatenate(outs, axis=1)
    out = jnp.dot(upd * gate, wo_ref[...], preferred_element_type=f32)
    out_ref[...] = out + bo_ref[...]


def _main(gathered, edge_features, rrep, trep, neg, ea, eb, ln_gamma,
          ln_beta, W_q, b_q, W_kv, b_kv, W_gate, b_gate, W_bg, b_bg, W_db, b_db, W_tb,
          W_out, b_out):
    full = lambda shape: pl.BlockSpec(shape, lambda i: tuple(0 for _ in shape))
    return pl.pallas_call(
        _main_body,
        grid=(N_BLOCKS,),
        in_specs=[
            pl.BlockSpec((BLK_E, TBL_W), lambda i: (i, 0)),
            pl.BlockSpec((BLK_E, C_Z), lambda i: (i, 0)),
            full((BLK_E * DEG, BLK_E)), full((BLK_E * DEG, BLK_E)),
            full((BLK_E, BLK_E)),
            full((24, 67)), full((24, 67)),
            full((1, C_Z)), full((1, C_Z)),
            full((C_Z, C_Z)), full((1, C_Z)),
            full((C_Z, 2 * C_Z)), full((1, 2 * C_Z)),
            full((C_Z, C_Z)), full((1, C_Z)),
            full((64, C_Z)), full((1, C_Z)),
            full((NUM_RBF, C_Z)), full((1, C_Z)),
            full((C_Z, H)),
            full((C_Z, C_Z)), full((1, C_Z)),
        ],
        out_specs=pl.BlockSpec((BLK_E, C_Z), lambda i: (i, 0)),
        out_shape=jax.ShapeDtypeStruct((E_EDGES, C_Z), jnp.float32),
    )(gathered, edge_features, rrep, trep, neg, ea, eb, ln_gamma, ln_beta,
      W_q, b_q,
      W_kv, b_kv, W_gate, b_gate, W_bg, b_bg, W_db, b_db, W_tb, W_out, b_out)


def kernel(node_features, node_trans, edge_features, edge_index, k, ln_gamma,
           ln_beta, W_nl, b_nl, W_nr, b_nr, W_bg, b_bg, W_db, b_db, W_tb,
           W_q, b_q, W_kv, b_kv, W_gate, b_gate, W_out, b_out):
    W_lr = jnp.concatenate([W_nl, W_nr], axis=1)
    b_lr = jnp.concatenate([b_nl, b_nr])[None, :]
    table = _build_table(node_trans, node_features, W_lr, b_lr)
    src = jnp.pad(edge_index[0].astype(jnp.int32), (0, E_PAD - E_EDGES))
    gathered = _sc_gather(table, src)
    p_ = np.arange(BLK_E * DEG)
    rrep = jnp.asarray((p_[:, None] // DEG) == np.arange(BLK_E)[None, :],
                       dtype=jnp.float32)
    tcol = (p_ // (DEG * DEG)) * DEG + (p_ % DEG)
    trep = jnp.asarray(tcol[:, None] == np.arange(BLK_E)[None, :],
                       dtype=jnp.float32)
    gsame = (np.arange(BLK_E)[:, None] // DEG) == (np.arange(BLK_E)[None, :]
                                                   // DEG)
    neg = jnp.asarray(np.where(gsame, 0.0, -np.inf), dtype=jnp.float32)
    ea_np = np.zeros((24, 67), np.float32)
    eb_np = np.zeros((24, 67), np.float32)
    for u in range(8):
        ea_np[8 + u, u * 8:(u + 1) * 8] = 1.0
        eb_np[16 + u, u:64:8] = 1.0
    for c in range(3):
        ea_np[c, 64 + c] = 1.0
        eb_np[c, 64 + c] = 1.0
    ea = jnp.asarray(ea_np)
    eb = jnp.asarray(eb_np)
    scale = jnp.float32(1.0 / np.sqrt(C_Z))
    row = lambda x: x[None, :]
    return _main(gathered, edge_features, rrep, trep, neg, ea, eb,
                 row(ln_gamma),
                 row(ln_beta), W_q * scale, row(b_q * scale), W_kv, row(b_kv),
                 W_gate, row(b_gate), W_bg, row(b_bg), W_db, row(b_db), W_tb,
                 W_out, row(b_out))


# speedup vs baseline: 191.9340x; 1.1239x over previous
import functools

import jax
import jax.numpy as jnp
import numpy as np
from jax import lax
from jax.experimental import pallas as pl
from jax.experimental.pallas import tpu as pltpu
from jax.experimental.pallas import tpu_sc as plsc

N_NODES = 10000
DEG = 16
E_EDGES = N_NODES * DEG
C_Z = 128
H = 4
D_H = C_Z // H
NUM_RBF = 64
TBL_W = 128

G = 16
BLK_E = G * DEG
N_BLOCKS = E_EDGES // BLK_E

SC_NC = 2
SC_NS = 16
SC_W = SC_NC * SC_NS
SC_CHUNK = 128
SC_CPW = 40
E_PAD = SC_W * SC_CPW * SC_CHUNK


def _table_body(nt_ref, nf_ref, w_ref, b_ref, out_ref):
    nf = nf_ref[...]
    nlr = jnp.dot(nf, w_ref[...], preferred_element_type=jnp.float32) + b_ref[...]
    nt = nt_ref[...]
    z5 = jnp.zeros((nt.shape[0], 5), jnp.float32)
    zpad = jnp.zeros((nt.shape[0], TBL_W - 24), jnp.float32)
    out_ref[...] = jnp.concatenate([nt, z5, nlr, zpad], axis=1)


def _build_table(node_trans, node_features, W_lr, b_lr):
    blk = 2000
    grid = (N_NODES // blk,)
    return pl.pallas_call(
        _table_body,
        grid=grid,
        in_specs=[
            pl.BlockSpec((blk, 3), lambda i: (i, 0)),
            pl.BlockSpec((blk, C_Z), lambda i: (i, 0)),
            pl.BlockSpec((C_Z, 16), lambda i: (0, 0)),
            pl.BlockSpec((1, 16), lambda i: (0, 0)),
        ],
        out_specs=pl.BlockSpec((blk, TBL_W), lambda i: (i, 0)),
        out_shape=jax.ShapeDtypeStruct((N_NODES, TBL_W), jnp.float32),
    )(node_trans, node_features, W_lr, b_lr)


def _sc_gather_body(table_hbm, idx_hbm, out_hbm, idx_v, rows_v, sem):
    wid = lax.axis_index("s") * SC_NC + lax.axis_index("c")

    def chunk(c, carry):
        base = wid * (SC_CPW * SC_CHUNK) + c * SC_CHUNK
        pltpu.sync_copy(idx_hbm.at[pl.ds(base, SC_CHUNK)], idx_v)
        pltpu.async_copy(table_hbm.at[idx_v], rows_v, sem).wait()
        pltpu.sync_copy(rows_v, out_hbm.at[pl.ds(base, SC_CHUNK)])
        return carry

    lax.fori_loop(0, SC_CPW, chunk, 0)


def _sc_gather(table, idx_pad):
    f = functools.partial(
        pl.kernel,
        out_type=jax.ShapeDtypeStruct((E_PAD, TBL_W), jnp.float32),
        mesh=plsc.VectorSubcoreMesh(
            core_axis_name="c", subcore_axis_name="s", num_cores=SC_NC),
        scratch_types=[
            pltpu.VMEM((SC_CHUNK,), jnp.int32),
            pltpu.VMEM((SC_CHUNK, TBL_W), jnp.float32),
            pltpu.SemaphoreType.DMA,
        ],
    )(_sc_gather_body)
    return f(table, idx_pad)


def _main_body(gat_ref, ef_ref, rrep_ref, trep_ref, neg_ref, ea_ref, eb_ref,
               lng_ref,
               lnb_ref, wq_ref, bq_ref, wkv_ref, bkv_ref, wg_ref, bg_ref,
               wbg_ref, bbg_ref, wdb_ref, bdb_ref, wtb_ref, wo_ref, bo_ref,
               out_ref):
    f32 = jnp.float32
    ef0 = ef_ref[...]
    mu_ = jnp.mean(ef0, axis=-1, keepdims=True)
    xc = ef0 - mu_
    var = jnp.mean(xc * xc, axis=-1, keepdims=True)
    ef = xc / jnp.sqrt(var + 1e-5) * lng_ref[...] + lnb_ref[...]

    q = jnp.dot(ef, wq_ref[...], preferred_element_type=f32) + bq_ref[...]
    kv = jnp.dot(ef, wkv_ref[...], preferred_element_type=f32) + bkv_ref[...]
    kk = kv[:, :C_Z]
    v = kv[:, C_Z:]
    gate = jax.nn.sigmoid(
        jnp.dot(ef, wg_ref[...], preferred_element_type=f32) + bg_ref[...])

    gat = gat_ref[...]

    def nbr16(col):
        cg = col.reshape(G, DEG)
        return jnp.broadcast_to(cg[:, None, :], (G, DEG, DEG)).reshape(
            BLK_E, DEG)

    tx, ty, tz = gat[:, 0:1], gat[:, 1:2], gat[:, 2:3]
    dx = tx - nbr16(tx)
    dy = ty - nbr16(ty)
    dz = tz - nbr16(tz)
    d2 = dx * dx + dy * dy + dz * dz
    rowm = lax.broadcasted_iota(jnp.int32, (BLK_E, DEG), 0) % DEG
    lane = lax.broadcasted_iota(jnp.int32, (BLK_E, DEG), 1)
    d2 = jnp.where(rowm == lane, jnp.inf, d2)

    d2_i = d2[:, None, :]
    d2_l = d2[:, :, None]
    li = lax.broadcasted_iota(jnp.int32, (1, DEG, DEG), 1)
    ii = lax.broadcasted_iota(jnp.int32, (1, DEG, DEG), 2)
    beats = (d2_l < d2_i) | ((d2_l == d2_i) & (li < ii))
    cnt = jnp.sum(beats.astype(f32), axis=1)
    sel128 = jnp.tile(cnt, (1, G)) < 8

    gat24 = gat[:, 0:24]
    lhsA = jnp.dot(gat24, ea_ref[...], preferred_element_type=f32)
    lhsB = jnp.dot(gat24, eb_ref[...], preferred_element_type=f32)
    aside = jnp.dot(rrep_ref[...], lhsA, preferred_element_type=f32)
    bside = jnp.dot(trep_ref[...], lhsB, preferred_element_type=f32)
    pair = aside[:, :64] * bside[:, :64]
    e3g = jnp.dot(pair, wbg_ref[...], preferred_element_type=f32) + bbg_ref[...]

    px = (aside[:, 64:65] - bside[:, 64:65]) + 1e-8
    py = (aside[:, 65:66] - bside[:, 65:66]) + 1e-8
    pz = (aside[:, 66:67] - bside[:, 66:67]) + 1e-8
    d1 = jnp.sqrt(px * px + py * py + pz * pz)
    mu_rbf = lax.broadcasted_iota(jnp.int32, (1, NUM_RBF), 1).astype(f32) * (
        20.0 / (NUM_RBF - 1))
    sigma = 20.0 / NUM_RBF
    rbf = jnp.exp(-(((d1 - mu_rbf) / sigma) ** 2))
    db = jnp.dot(rbf, wdb_ref[...], preferred_element_type=f32) + bdb_ref[...]

    zb = jax.nn.sigmoid(e3g) * db
    bias4 = jnp.dot(zb, wtb_ref[...], preferred_element_type=f32)
    b64h = jnp.swapaxes(bias4.reshape(BLK_E, DEG, H), 1, 2).reshape(
        BLK_E, DEG * H)

    neg = neg_ref[...]
    outs = []
    for h in range(H):
        qh = q[:, h * D_H:(h + 1) * D_H]
        kh = kk[:, h * D_H:(h + 1) * D_H]
        vh = v[:, h * D_H:(h + 1) * D_H]
        lg = lax.dot_general(qh, kh, (((1,), (1,)), ((), ())),
                             preferred_element_type=f32)
        lg = lg + jnp.tile(b64h[:, h * DEG:(h + 1) * DEG], (1, G)) + neg
        lg = jnp.where(sel128, lg, -jnp.inf)
        mx = jnp.max(lg, axis=-1, keepdims=True)
        ex = jnp.exp(lg - mx)
        sm = jnp.sum(ex, axis=-1, keepdims=True)
        p = ex / (sm + 1e-16)
        outs.append(jnp.dot(p, vh, preferred_element_type=f32))
    upd = jnp.concatenate(outs, axis=1)
    out = jnp.dot(upd * gate, wo_ref[...], preferred_element_type=f32)
    out_ref[...] = out + bo_ref[...]


def _main(gathered, edge_features, rrep, trep, neg, ea, eb, ln_gamma,
          ln_beta, W_q, b_q, W_kv, b_kv, W_gate, b_gate, W_bg, b_bg, W_db, b_db, W_tb,
          W_out, b_out):
    full = lambda shape: pl.BlockSpec(shape, lambda i: tuple(0 for _ in shape))
    return pl.pallas_call(
        _main_body,
        grid=(N_BLOCKS,),
        in_specs=[
            pl.BlockSpec((BLK_E, TBL_W), lambda i: (i, 0)),
            pl.BlockSpec((BLK_E, C_Z), lambda i: (i, 0)),
            full((BLK_E * DEG, BLK_E)), full((BLK_E * DEG, BLK_E)),
            full((BLK_E, BLK_E)),
            full((24, 67)), full((24, 67)),
            full((1, C_Z)), full((1, C_Z)),
            full((C_Z, C_Z)), full((1, C_Z)),
            full((C_Z, 2 * C_Z)), full((1, 2 * C_Z)),
            full((C_Z, C_Z)), full((1, C_Z)),
            full((64, C_Z)), full((1, C_Z)),
            full((NUM_RBF, C_Z)), full((1, C_Z)),
            full((C_Z, H)),
            full((C_Z, C_Z)), full((1, C_Z)),
        ],
        out_specs=pl.BlockSpec((BLK_E, C_Z), lambda i: (i, 0)),
        out_shape=jax.ShapeDtypeStruct((E_EDGES, C_Z), jnp.float32),
    )(gathered, edge_features, rrep, trep, neg, ea, eb, ln_gamma, ln_beta,
      W_q, b_q,
      W_kv, b_kv, W_gate, b_gate, W_bg, b_bg, W_db, b_db, W_tb, W_out, b_out)


def kernel(node_features, node_trans, edge_features, edge_index, k, ln_gamma,
           ln_beta, W_nl, b_nl, W_nr, b_nr, W_bg, b_bg, W_db, b_db, W_tb,
           W_q, b_q, W_kv, b_kv, W_gate, b_gate, W_out, b_out):
    W_lr = jnp.concatenate([W_nl, W_nr], axis=1)
    b_lr = jnp.concatenate([b_nl, b_nr])[None, :]
    table = _build_table(node_trans, node_features, W_lr, b_lr)
    src = jnp.pad(edge_index[0].astype(jnp.int32), (0, E_PAD - E_EDGES))
    gathered = _sc_gather(table, src)
    p_ = np.arange(BLK_E * DEG)
    rrep = jnp.asarray((p_[:, None] // DEG) == np.arange(BLK_E)[None, :],
                       dtype=jnp.float32)
    tcol = (p_ // (DEG * DEG)) * DEG + (p_ % DEG)
    trep = jnp.asarray(tcol[:, None] == np.arange(BLK_E)[None, :],
                       dtype=jnp.float32)
    gsame = (np.arange(BLK_E)[:, None] // DEG) == (np.arange(BLK_E)[None, :]
                                                   // DEG)
    neg = jnp.asarray(np.where(gsame, 0.0, -np.inf), dtype=jnp.float32)
    ea_np = np.zeros((24, 67), np.float32)
    eb_np = np.zeros((24, 67), np.float32)
    for u in range(8):
        ea_np[8 + u, u * 8:(u + 1) * 8] = 1.0
        eb_np[16 + u, u:64:8] = 1.0
    for c in range(3):
        ea_np[c, 64 + c] = 1.0
        eb_np[c, 64 + c] = 1.0
    ea = jnp.asarray(ea_np)
    eb = jnp.asarray(eb_np)
    scale = jnp.float32(1.0 / np.sqrt(C_Z))
    row = lambda x: x[None, :]
    return _main(gathered, edge_features, rrep, trep, neg, ea, eb,
                 row(ln_gamma),
                 row(ln_beta), W_q * scale, row(b_q * scale), W_kv, row(b_kv),
                 W_gate, row(b_gate), W_bg, row(b_bg), W_db, row(b_db), W_tb,
                 W_out, row(b_out))


# speedup vs baseline: 194.5388x; 1.0136x over previous
import functools

import jax
import jax.numpy as jnp
import numpy as np
from jax import lax
from jax.experimental import pallas as pl
from jax.experimental.pallas import tpu as pltpu
from jax.experimental.pallas import tpu_sc as plsc

N_NODES = 10000
DEG = 16
E_EDGES = N_NODES * DEG
C_Z = 128
H = 4
D_H = C_Z // H
NUM_RBF = 64
TBL_W = 128

G = 16
BLK_E = G * DEG
N_BLOCKS = E_EDGES // BLK_E

SC_NC = 2
SC_NS = 16
SC_W = SC_NC * SC_NS
SC_CHUNK = 128
SC_CPW = 20
E_HALF = SC_W * SC_CPW * SC_CHUNK
E_PAD = 2 * E_HALF


def _table_body(nt_ref, nf_ref, w_ref, b_ref, out_ref):
    nf = nf_ref[...]
    nlr = jnp.dot(nf, w_ref[...], preferred_element_type=jnp.float32) + b_ref[...]
    nt = nt_ref[...]
    one = jnp.ones((nt.shape[0], 1), jnp.float32)
    z4 = jnp.zeros((nt.shape[0], 4), jnp.float32)
    zpad = jnp.zeros((nt.shape[0], TBL_W - 24), jnp.float32)
    out_ref[...] = jnp.concatenate([nt, one, z4, nlr, zpad], axis=1)


def _build_table(node_trans, node_features, W_lr, b_lr):
    blk = 2000
    grid = (N_NODES // blk,)
    return pl.pallas_call(
        _table_body,
        grid=grid,
        in_specs=[
            pl.BlockSpec((blk, 3), lambda i: (i, 0)),
            pl.BlockSpec((blk, C_Z), lambda i: (i, 0)),
            pl.BlockSpec((C_Z, 16), lambda i: (0, 0)),
            pl.BlockSpec((1, 16), lambda i: (0, 0)),
        ],
        out_specs=pl.BlockSpec((blk, TBL_W), lambda i: (i, 0)),
        out_shape=jax.ShapeDtypeStruct((N_NODES, TBL_W), jnp.float32),
    )(node_trans, node_features, W_lr, b_lr)


def _sc_gather_body(table_hbm, idx_hbm, out_hbm, idx_v, rows_v, sem):
    wid = lax.axis_index("s") * SC_NC + lax.axis_index("c")

    def chunk(c, carry):
        base = wid * (SC_CPW * SC_CHUNK) + c * SC_CHUNK
        pltpu.sync_copy(idx_hbm.at[pl.ds(base, SC_CHUNK)], idx_v)
        pltpu.async_copy(table_hbm.at[idx_v], rows_v, sem).wait()
        pltpu.sync_copy(rows_v, out_hbm.at[pl.ds(base, SC_CHUNK)])
        return carry

    lax.fori_loop(0, SC_CPW, chunk, 0)


def _sc_gather(table, idx_pad):
    f = functools.partial(
        pl.kernel,
        out_type=jax.ShapeDtypeStruct((E_HALF, TBL_W), jnp.float32),
        mesh=plsc.VectorSubcoreMesh(
            core_axis_name="c", subcore_axis_name="s", num_cores=SC_NC),
        scratch_types=[
            pltpu.VMEM((SC_CHUNK,), jnp.int32),
            pltpu.VMEM((SC_CHUNK, TBL_W), jnp.float32),
            pltpu.SemaphoreType.DMA,
        ],
    )(_sc_gather_body)
    return f(table, idx_pad)


def _main_body(gat_ref, ef_ref, rrep_ref, trep_ref, neg_ref, ea_ref, eb_ref,
               lng_ref,
               lnb_ref, wq_ref, bq_ref, wkv_ref, bkv_ref, wg_ref, bg_ref,
               wbg_ref, bbg_ref, wdb_ref, bdb_ref, wtb_ref, wo_ref, bo_ref,
               out_ref):
    f32 = jnp.float32
    ef0 = ef_ref[...]
    mu_ = jnp.mean(ef0, axis=-1, keepdims=True)
    xc = ef0 - mu_
    var = jnp.mean(xc * xc, axis=-1, keepdims=True)
    ef = xc / jnp.sqrt(var + 1e-5) * lng_ref[...] + lnb_ref[...]

    q = jnp.dot(ef, wq_ref[...], preferred_element_type=f32) + bq_ref[...]
    kv = jnp.dot(ef, wkv_ref[...], preferred_element_type=f32) + bkv_ref[...]
    kk = kv[:, :C_Z]
    v = kv[:, C_Z:]
    gate = jax.nn.sigmoid(
        jnp.dot(ef, wg_ref[...], preferred_element_type=f32) + bg_ref[...])

    gat = gat_ref[...]

    def nbr16(col):
        cg = col.reshape(G, DEG)
        return jnp.broadcast_to(cg[:, None, :], (G, DEG, DEG)).reshape(
            BLK_E, DEG)

    tx, ty, tz = gat[:, 0:1], gat[:, 1:2], gat[:, 2:3]
    dx = tx - nbr16(tx)
    dy = ty - nbr16(ty)
    dz = tz - nbr16(tz)
    d2 = dx * dx + dy * dy + dz * dz
    rowm = lax.broadcasted_iota(jnp.int32, (BLK_E, DEG), 0) % DEG
    lane = lax.broadcasted_iota(jnp.int32, (BLK_E, DEG), 1)
    d2 = jnp.where(rowm == lane, jnp.inf, d2)

    d2_i = d2[:, None, :]
    d2_l = d2[:, :, None]
    li = lax.broadcasted_iota(jnp.int32, (1, DEG, DEG), 1)
    ii = lax.broadcasted_iota(jnp.int32, (1, DEG, DEG), 2)
    beats = (d2_l < d2_i) | ((d2_l == d2_i) & (li < ii))
    cnt = jnp.sum(beats.astype(f32), axis=1)
    sel128 = jnp.tile(cnt, (1, G)) < 8

    gat24 = gat[:, 0:24]
    lhsA = jnp.dot(gat24, ea_ref[...], preferred_element_type=f32)
    lhsB = jnp.dot(gat24, eb_ref[...], preferred_element_type=f32)
    aside = jnp.dot(rrep_ref[...], lhsA, preferred_element_type=f32)
    bside = jnp.dot(trep_ref[...], lhsB, preferred_element_type=f32)
    pair = aside[:, :64] * bside[:, :64]
    e3g = jnp.dot(pair, wbg_ref[...], preferred_element_type=f32) + bbg_ref[...]

    pq = aside[:, 64:67] - bside[:, 64:67]
    sq = pq * pq
    d1 = jnp.sqrt(sq[:, 0:1] + sq[:, 1:2] + sq[:, 2:3])
    mu_rbf = lax.broadcasted_iota(jnp.int32, (1, NUM_RBF), 1).astype(f32) * (
        20.0 / (NUM_RBF - 1))
    sigma = 20.0 / NUM_RBF
    rbf = jnp.exp(-(((d1 - mu_rbf) / sigma) ** 2))
    db = jnp.dot(rbf, wdb_ref[...], preferred_element_type=f32) + bdb_ref[...]

    zb = jax.nn.sigmoid(e3g) * db
    bias4 = jnp.dot(zb, wtb_ref[...], preferred_element_type=f32)
    b64h = jnp.swapaxes(bias4.reshape(BLK_E, DEG, H), 1, 2).reshape(
        BLK_E, DEG * H)

    neg = neg_ref[...]
    outs = []
    for h in range(H):
        qh = q[:, h * D_H:(h + 1) * D_H]
        kh = kk[:, h * D_H:(h + 1) * D_H]
        vh = v[:, h * D_H:(h + 1) * D_H]
        lg = lax.dot_general(qh, kh, (((1,), (1,)), ((), ())),
                             preferred_element_type=f32)
        lg = lg + jnp.tile(b64h[:, h * DEG:(h + 1) * DEG], (1, G)) + neg
        lg = jnp.where(sel128, lg, -jnp.inf)
        mx = jnp.max(lg, axis=-1, keepdims=True)
        ex = jnp.exp(lg - mx)
        sm = jnp.sum(ex, axis=-1, keepdims=True)
        p = ex / (sm + 1e-16)
        outs.append(jnp.dot(p, vh, preferred_element_type=f32))
    upd = jnp.concatenate(outs, axis=1)
    out = jnp.dot(upd * gate, wo_ref[...], preferred_element_type=f32)
    out_ref[...] = out + bo_ref[...]


def _main(gathered, edge_features, rrep, trep, neg, ea, eb, ln_gamma,
          ln_beta, W_q, b_q, W_kv, b_kv, W_gate, b_gate, W_bg, b_bg, W_db, b_db, W_tb,
          W_out, b_out, n_blocks, ef_off, out_rows):
    full = lambda shape: pl.BlockSpec(shape, lambda i: tuple(0 for _ in shape))
    return pl.pallas_call(
        _main_body,
        grid=(n_blocks,),
        in_specs=[
            pl.BlockSpec((BLK_E, TBL_W), lambda i: (i, 0)),
            pl.BlockSpec((BLK_E, C_Z), lambda i: (i + ef_off, 0)),
            full((BLK_E * DEG, BLK_E)), full((BLK_E * DEG, BLK_E)),
            full((BLK_E, BLK_E)),
            full((24, 67)), full((24, 67)),
            full((1, C_Z)), full((1, C_Z)),
            full((C_Z, C_Z)), full((1, C_Z)),
            full((C_Z, 2 * C_Z)), full((1, 2 * C_Z)),
            full((C_Z, C_Z)), full((1, C_Z)),
            full((64, C_Z)), full((1, C_Z)),
            full((NUM_RBF, C_Z)), full((1, C_Z)),
            full((C_Z, H)),
            full((C_Z, C_Z)), full((1, C_Z)),
        ],
        out_specs=pl.BlockSpec((BLK_E, C_Z), lambda i: (i, 0)),
        out_shape=jax.ShapeDtypeStruct((out_rows, C_Z), jnp.float32),
    )(gathered, edge_features, rrep, trep, neg, ea, eb, ln_gamma, ln_beta,
      W_q, b_q,
      W_kv, b_kv, W_gate, b_gate, W_bg, b_bg, W_db, b_db, W_tb, W_out, b_out)


def kernel(node_features, node_trans, edge_features, edge_index, k, ln_gamma,
           ln_beta, W_nl, b_nl, W_nr, b_nr, W_bg, b_bg, W_db, b_db, W_tb,
           W_q, b_q, W_kv, b_kv, W_gate, b_gate, W_out, b_out):
    W_lr = jnp.concatenate([W_nl, W_nr], axis=1)
    b_lr = jnp.concatenate([b_nl, b_nr])[None, :]
    table = _build_table(node_trans, node_features, W_lr, b_lr)
    src = jnp.pad(edge_index[0].astype(jnp.int32), (0, E_PAD - E_EDGES))
    g1 = _sc_gather(table, src[0:E_HALF])
    g2 = _sc_gather(table, src[E_HALF:E_PAD])
    p_ = np.arange(BLK_E * DEG)
    rrep = jnp.asarray((p_[:, None] // DEG) == np.arange(BLK_E)[None, :],
                       dtype=jnp.float32)
    tcol = (p_ // (DEG * DEG)) * DEG + (p_ % DEG)
    trep = jnp.asarray(tcol[:, None] == np.arange(BLK_E)[None, :],
                       dtype=jnp.float32)
    gsame = (np.arange(BLK_E)[:, None] // DEG) == (np.arange(BLK_E)[None, :]
                                                   // DEG)
    neg = jnp.asarray(np.where(gsame, 0.0, -np.inf), dtype=jnp.float32)
    ea_np = np.zeros((24, 67), np.float32)
    eb_np = np.zeros((24, 67), np.float32)
    for u in range(8):
        ea_np[8 + u, u * 8:(u + 1) * 8] = 1.0
        eb_np[16 + u, u:64:8] = 1.0
    for c in range(3):
        ea_np[c, 64 + c] = 1.0
        eb_np[c, 64 + c] = 1.0
    ea_np[3, 64:67] = 1e-8
    ea = jnp.asarray(ea_np)
    eb = jnp.asarray(eb_np)
    scale = jnp.float32(1.0 / np.sqrt(C_Z))
    row = lambda x: x[None, :]
    wargs = (rrep, trep, neg, ea, eb, row(ln_gamma), row(ln_beta),
             W_q * scale, row(b_q * scale), W_kv, row(b_kv), W_gate,
             row(b_gate), W_bg, row(b_bg), W_db, row(b_db), W_tb, W_out,
             row(b_out))
    nb1 = E_HALF // BLK_E
    nb2 = (E_EDGES - E_HALF) // BLK_E
    out1 = _main(g1, edge_features, *wargs, n_blocks=nb1, ef_off=0,
                 out_rows=E_HALF)
    out2 = _main(g2, edge_features, *wargs, n_blocks=nb2, ef_off=nb1,
                 out_rows=E_EDGES - E_HALF)
    return jnp.concatenate([out1, out2], axis=0)


# speedup vs baseline: 194.6768x; 1.0007x over previous
import functools

import jax
import jax.numpy as jnp
import numpy as np
from jax import lax
from jax.experimental import pallas as pl
from jax.experimental.pallas import tpu as pltpu
from jax.experimental.pallas import tpu_sc as plsc

N_NODES = 10000
DEG = 16
E_EDGES = N_NODES * DEG
C_Z = 128
H = 4
D_H = C_Z // H
NUM_RBF = 64
TBL_W = 128

G = 16
BLK_E = G * DEG
N_BLOCKS = E_EDGES // BLK_E

SC_NC = 2
SC_NS = 16
SC_W = SC_NC * SC_NS
SC_CHUNK = 128
SC_CPW = 10
E_PART = SC_W * SC_CPW * SC_CHUNK
N_PARTS = 4
E_PAD = N_PARTS * E_PART


def _table_body(nt_ref, nf_ref, w_ref, b_ref, out_ref):
    nf = nf_ref[...]
    nlr = jnp.dot(nf, w_ref[...], preferred_element_type=jnp.float32) + b_ref[...]
    nt = nt_ref[...]
    one = jnp.ones((nt.shape[0], 1), jnp.float32)
    z4 = jnp.zeros((nt.shape[0], 4), jnp.float32)
    zpad = jnp.zeros((nt.shape[0], TBL_W - 24), jnp.float32)
    out_ref[...] = jnp.concatenate([nt, one, z4, nlr, zpad], axis=1)


def _build_table(node_trans, node_features, W_lr, b_lr):
    blk = 2000
    grid = (N_NODES // blk,)
    return pl.pallas_call(
        _table_body,
        grid=grid,
        in_specs=[
            pl.BlockSpec((blk, 3), lambda i: (i, 0)),
            pl.BlockSpec((blk, C_Z), lambda i: (i, 0)),
            pl.BlockSpec((C_Z, 16), lambda i: (0, 0)),
            pl.BlockSpec((1, 16), lambda i: (0, 0)),
        ],
        out_specs=pl.BlockSpec((blk, TBL_W), lambda i: (i, 0)),
        out_shape=jax.ShapeDtypeStruct((N_NODES, TBL_W), jnp.float32),
    )(node_trans, node_features, W_lr, b_lr)


def _sc_gather_body(table_hbm, idx_hbm, out_hbm, idx_v, rows_v, sem):
    wid = lax.axis_index("s") * SC_NC + lax.axis_index("c")

    def chunk(c, carry):
        base = wid * (SC_CPW * SC_CHUNK) + c * SC_CHUNK
        pltpu.sync_copy(idx_hbm.at[pl.ds(base, SC_CHUNK)], idx_v)
        pltpu.async_copy(table_hbm.at[idx_v], rows_v, sem).wait()
        pltpu.sync_copy(rows_v, out_hbm.at[pl.ds(base, SC_CHUNK)])
        return carry

    lax.fori_loop(0, SC_CPW, chunk, 0)


def _sc_gather(table, idx_pad):
    f = functools.partial(
        pl.kernel,
        out_type=jax.ShapeDtypeStruct((E_PART, TBL_W), jnp.float32),
        mesh=plsc.VectorSubcoreMesh(
            core_axis_name="c", subcore_axis_name="s", num_cores=SC_NC),
        scratch_types=[
            pltpu.VMEM((SC_CHUNK,), jnp.int32),
            pltpu.VMEM((SC_CHUNK, TBL_W), jnp.float32),
            pltpu.SemaphoreType.DMA,
        ],
    )(_sc_gather_body)
    return f(table, idx_pad)


def _main_body(gat_ref, ef_ref, rrep_ref, trep_ref, neg_ref, ea_ref, eb_ref,
               lng_ref,
               lnb_ref, wq_ref, bq_ref, wkv_ref, bkv_ref, wg_ref, bg_ref,
               wbg_ref, bbg_ref, wdb_ref, bdb_ref, wtb_ref, wo_ref, bo_ref,
               out_ref):
    f32 = jnp.float32
    ef0 = ef_ref[...]
    mu_ = jnp.mean(ef0, axis=-1, keepdims=True)
    xc = ef0 - mu_
    var = jnp.mean(xc * xc, axis=-1, keepdims=True)
    ef = xc / jnp.sqrt(var + 1e-5) * lng_ref[...] + lnb_ref[...]

    q = jnp.dot(ef, wq_ref[...], preferred_element_type=f32) + bq_ref[...]
    kv = jnp.dot(ef, wkv_ref[...], preferred_element_type=f32) + bkv_ref[...]
    kk = kv[:, :C_Z]
    v = kv[:, C_Z:]
    gate = jax.nn.sigmoid(
        jnp.dot(ef, wg_ref[...], preferred_element_type=f32) + bg_ref[...])

    gat = gat_ref[...]

    def nbr16(col):
        cg = col.reshape(G, DEG)
        return jnp.broadcast_to(cg[:, None, :], (G, DEG, DEG)).reshape(
            BLK_E, DEG)

    tx, ty, tz = gat[:, 0:1], gat[:, 1:2], gat[:, 2:3]
    dx = tx - nbr16(tx)
    dy = ty - nbr16(ty)
    dz = tz - nbr16(tz)
    d2 = dx * dx + dy * dy + dz * dz
    rowm = lax.broadcasted_iota(jnp.int32, (BLK_E, DEG), 0) % DEG
    lane = lax.broadcasted_iota(jnp.int32, (BLK_E, DEG), 1)
    d2 = jnp.where(rowm == lane, jnp.inf, d2)

    d2_i = d2[:, None, :]
    d2_l = d2[:, :, None]
    li = lax.broadcasted_iota(jnp.int32, (1, DEG, DEG), 1)
    ii = lax.broadcasted_iota(jnp.int32, (1, DEG, DEG), 2)
    beats = (d2_l < d2_i) | ((d2_l == d2_i) & (li < ii))
    cnt = jnp.sum(beats.astype(f32), axis=1)
    sel128 = jnp.tile(cnt, (1, G)) < 8

    gat24 = gat[:, 0:24]
    lhsA = jnp.dot(gat24, ea_ref[...], preferred_element_type=f32)
    lhsB = jnp.dot(gat24, eb_ref[...], preferred_element_type=f32)
    aside = jnp.dot(rrep_ref[...], lhsA, preferred_element_type=f32)
    bside = jnp.dot(trep_ref[...], lhsB, preferred_element_type=f32)
    pair = aside[:, :64] * bside[:, :64]
    e3g = jnp.dot(pair, wbg_ref[...], preferred_element_type=f32) + bbg_ref[...]

    pq = aside[:, 64:67] - bside[:, 64:67]
    sq = pq * pq
    d1 = jnp.sqrt(sq[:, 0:1] + sq[:, 1:2] + sq[:, 2:3])
    mu_rbf = lax.broadcasted_iota(jnp.int32, (1, NUM_RBF), 1).astype(f32) * (
        20.0 / (NUM_RBF - 1))
    sigma = 20.0 / NUM_RBF
    rbf = jnp.exp(-(((d1 - mu_rbf) / sigma) ** 2))
    db = jnp.dot(rbf, wdb_ref[...], preferred_element_type=f32) + bdb_ref[...]

    zb = jax.nn.sigmoid(e3g) * db
    bias4 = jnp.dot(zb, wtb_ref[...], preferred_element_type=f32)
    b64h = jnp.swapaxes(bias4.reshape(BLK_E, DEG, H), 1, 2).reshape(
        BLK_E, DEG * H)

    neg = neg_ref[...]
    outs = []
    for h in range(H):
        qh = q[:, h * D_H:(h + 1) * D_H]
        kh = kk[:, h * D_H:(h + 1) * D_H]
        vh = v[:, h * D_H:(h + 1) * D_H]
        lg = lax.dot_general(qh, kh, (((1,), (1,)), ((), ())),
                             preferred_element_type=f32)
        lg = lg + jnp.tile(b64h[:, h * DEG:(h + 1) * DEG], (1, G)) + neg
        lg = jnp.where(sel128, lg, -jnp.inf)
        mx = jnp.max(lg, axis=-1, keepdims=True)
        ex = jnp.exp(lg - mx)
        sm = jnp.sum(ex, axis=-1, keepdims=True)
        p = ex / (sm + 1e-16)
        outs.append(jnp.dot(p, vh, preferred_element_type=f32))
    upd = jnp.concatenate(outs, axis=1)
    out = jnp.dot(upd * gate, wo_ref[...], preferred_element_type=f32)
    out_ref[...] = out + bo_ref[...]


def _main(gathered, edge_features, rrep, trep, neg, ea, eb, ln_gamma,
          ln_beta, W_q, b_q, W_kv, b_kv, W_gate, b_gate, W_bg, b_bg, W_db, b_db, W_tb,
          W_out, b_out, n_blocks, ef_off, out_rows):
    full = lambda shape: pl.BlockSpec(shape, lambda i: tuple(0 for _ in shape))
    return pl.pallas_call(
        _main_body,
        grid=(n_blocks,),
        in_specs=[
            pl.BlockSpec((BLK_E, TBL_W), lambda i: (i, 0)),
            pl.BlockSpec((BLK_E, C_Z), lambda i: (i + ef_off, 0)),
            full((BLK_E * DEG, BLK_E)), full((BLK_E * DEG, BLK_E)),
            full((BLK_E, BLK_E)),
            full((24, 67)), full((24, 67)),
            full((1, C_Z)), full((1, C_Z)),
            full((C_Z, C_Z)), full((1, C_Z)),
            full((C_Z, 2 * C_Z)), full((1, 2 * C_Z)),
            full((C_Z, C_Z)), full((1, C_Z)),
            full((64, C_Z)), full((1, C_Z)),
            full((NUM_RBF, C_Z)), full((1, C_Z)),
            full((C_Z, H)),
            full((C_Z, C_Z)), full((1, C_Z)),
        ],
        out_specs=pl.BlockSpec((BLK_E, C_Z), lambda i: (i, 0)),
        out_shape=jax.ShapeDtypeStruct((out_rows, C_Z), jnp.float32),
    )(gathered, edge_features, rrep, trep, neg, ea, eb, ln_gamma, ln_beta,
      W_q, b_q,
      W_kv, b_kv, W_gate, b_gate, W_bg, b_bg, W_db, b_db, W_tb, W_out, b_out)


def kernel(node_features, node_trans, edge_features, edge_index, k, ln_gamma,
           ln_beta, W_nl, b_nl, W_nr, b_nr, W_bg, b_bg, W_db, b_db, W_tb,
           W_q, b_q, W_kv, b_kv, W_gate, b_gate, W_out, b_out):
    W_lr = jnp.concatenate([W_nl, W_nr], axis=1)
    b_lr = jnp.concatenate([b_nl, b_nr])[None, :]
    table = _build_table(node_trans, node_features, W_lr, b_lr)
    src = jnp.pad(edge_index[0].astype(jnp.int32), (0, E_PAD - E_EDGES))
    gs = [_sc_gather(table, src[i * E_PART:(i + 1) * E_PART])
          for i in range(N_PARTS)]
    p_ = np.arange(BLK_E * DEG)
    rrep = jnp.asarray((p_[:, None] // DEG) == np.arange(BLK_E)[None, :],
                       dtype=jnp.float32)
    tcol = (p_ // (DEG * DEG)) * DEG + (p_ % DEG)
    trep = jnp.asarray(tcol[:, None] == np.arange(BLK_E)[None, :],
                       dtype=jnp.float32)
    gsame = (np.arange(BLK_E)[:, None] // DEG) == (np.arange(BLK_E)[None, :]
                                                   // DEG)
    neg = jnp.asarray(np.where(gsame, 0.0, -np.inf), dtype=jnp.float32)
    ea_np = np.zeros((24, 67), np.float32)
    eb_np = np.zeros((24, 67), np.float32)
    for u in range(8):
        ea_np[8 + u, u * 8:(u + 1) * 8] = 1.0
        eb_np[16 + u, u:64:8] = 1.0
    for c in range(3):
        ea_np[c, 64 + c] = 1.0
        eb_np[c, 64 + c] = 1.0
    ea_np[3, 64:67] = 1e-8
    ea = jnp.asarray(ea_np)
    eb = jnp.asarray(eb_np)
    scale = jnp.float32(1.0 / np.sqrt(C_Z))
    row = lambda x: x[None, :]
    wargs = (rrep, trep, neg, ea, eb, row(ln_gamma), row(ln_beta),
             W_q * scale, row(b_q * scale), W_kv, row(b_kv), W_gate,
             row(b_gate), W_bg, row(b_bg), W_db, row(b_db), W_tb, W_out,
             row(b_out))
    nbp = E_PART // BLK_E
    outs = []
    for i in range(N_PARTS):
        rows = min(E_PART, E_EDGES - i * E_PART)
        outs.append(_main(gs[i], edge_features, *wargs,
                          n_blocks=rows // BLK_E, ef_off=i * nbp,
                          out_rows=rows))
    return jnp.concatenate(outs, axis=0)
